# baseline (speedup 1.0000x reference)
import jax
import jax.numpy as jnp
import numpy as np
from jax.experimental import pallas as pl

_N = 10000
_B = 32
_K = 16
_D_NODE = 128
_D_EDGE = 16
_D_FEAT = 32
_D_ENC = 32
_D_HID = 64
_R = 40


def _dot(a, b):
    return jax.lax.dot_general(a, b, (((1,), (0,)), ((), ())),
                               preferred_element_type=jnp.float32)


def _fused_kernel(root_ref, neigh_ref, edge_ref, rts_ref, nts_ref, nid_ref,
                  gum_ref, Wn_ref, bn_ref, We_ref, be_ref, Wq_ref, bq_ref,
                  Wk_ref, bk_ref, w_ref, e0_ref, e1_ref,
                  probs_ref, action_ref):
    R, B = _R, _B
    w = w_ref[0, :]

    rf = jnp.maximum(_dot(root_ref[...], Wn_ref[...]) + bn_ref[...], 0.0)
    root_enc = jnp.concatenate(
        [rf,
         jnp.broadcast_to(e0_ref[...], (R, _D_ENC)),
         jnp.broadcast_to(e1_ref[...], (R, _D_ENC))], axis=1)
    q = _dot(root_enc, Wq_ref[...]) + bq_ref[...]

    nf = jnp.maximum(_dot(neigh_ref[...].reshape(R * B, _D_NODE), Wn_ref[...])
                     + bn_ref[...], 0.0)
    ef = jnp.maximum(_dot(edge_ref[...].reshape(R * B, _D_EDGE), We_ref[...])
                     + be_ref[...], 0.0)
    dt = rts_ref[...] - nts_ref[...]
    te = jnp.cos(dt[:, :, None] * w[None, None, :])
    nid = nid_ref[...]
    mask = nid[:, :, None] == nid[:, None, :]
    maskf = mask.astype(jnp.float32)
    freq = mask.sum(axis=2).astype(jnp.float32) / float(B)
    fe = jnp.cos(freq[:, :, None] * w[None, None, :])
    link = jnp.concatenate(
        [nf.reshape(R, B, _D_FEAT), ef.reshape(R, B, _D_FEAT), te, fe, maskf],
        axis=2)
    k = _dot(link.reshape(R * B, 160), Wk_ref[...]) + bk_ref[...]
    k3 = k.reshape(R, B, _D_HID)

    scores = jnp.sum(q[:, None, :] * k3, axis=2) / np.sqrt(float(_D_HID))

    m = jnp.max(scores, axis=1, keepdims=True)
    e = jnp.exp(scores - m)
    probs = e / jnp.sum(e, axis=1, keepdims=True)
    probs_ref[...] = probs

    pert = jnp.log(probs + 1e-20) + gum_ref[...]
    pj = pert[:, None, :]
    pb = pert[:, :, None]
    jj = jax.lax.broadcasted_iota(jnp.int32, (R, B, B), 2)
    bb = jax.lax.broadcasted_iota(jnp.int32, (R, B, B), 1)
    beats = jnp.logical_or(pj > pb, jnp.logical_and(pj == pb, jj < bb))
    rank = jnp.sum(beats.astype(jnp.int32), axis=2)
    ss = jax.lax.broadcasted_iota(jnp.int32, (R, B, _K), 2)
    bsel = jax.lax.broadcasted_iota(jnp.int32, (R, B, _K), 1)
    onehot = (rank[:, :, None] == ss).astype(jnp.int32)
    action_ref[...] = jnp.sum(onehot * bsel, axis=1)


def kernel(root_node_feature, neighbor_node_feature, neighbor_edge_feature,
           root_ts, neighbor_ts, neighbor_nid,
           W_node, b_node, W_edge, b_edge, W_q, b_q, W_k, b_k):
    N, B = neighbor_nid.shape
    w = (1.0 / (10.0 ** jnp.linspace(0.0, 9.0, _D_ENC))).astype(jnp.float32)
    enc0 = jnp.cos(jnp.zeros((1,), jnp.float32)[:, None] * w[None, :])
    enc1 = jnp.cos(jnp.ones((1,), jnp.float32)[:, None] * w[None, :])

    gkey = jax.random.fold_in(jax.random.key(0), 123)
    u = jax.random.uniform(gkey, (N, B), jnp.float32, 1e-10, 1.0)
    gumbel = -jnp.log(-jnp.log(u))

    grid = (N // _R,)
    cst = lambda i: (0, 0)
    probs, action = pl.pallas_call(
        _fused_kernel,
        grid=grid,
        in_specs=[
            pl.BlockSpec((_R, _D_NODE), lambda i: (i, 0)),
            pl.BlockSpec((_R, B, _D_NODE), lambda i: (i, 0, 0)),
            pl.BlockSpec((_R, B, _D_EDGE), lambda i: (i, 0, 0)),
            pl.BlockSpec((_R, 1), lambda i: (i, 0)),
            pl.BlockSpec((_R, B), lambda i: (i, 0)),
            pl.BlockSpec((_R, B), lambda i: (i, 0)),
            pl.BlockSpec((_R, B), lambda i: (i, 0)),
            pl.BlockSpec((_D_NODE, _D_FEAT), cst),
            pl.BlockSpec((1, _D_FEAT), cst),
            pl.BlockSpec((_D_EDGE, _D_FEAT), cst),
            pl.BlockSpec((1, _D_FEAT), cst),
            pl.BlockSpec((96, _D_HID), cst),
            pl.BlockSpec((1, _D_HID), cst),
            pl.BlockSpec((160, _D_HID), cst),
            pl.BlockSpec((1, _D_HID), cst),
            pl.BlockSpec((1, _D_ENC), cst),
            pl.BlockSpec((1, _D_ENC), cst),
            pl.BlockSpec((1, _D_ENC), cst),
        ],
        out_specs=[
            pl.BlockSpec((_R, B), lambda i: (i, 0)),
            pl.BlockSpec((_R, _K), lambda i: (i, 0)),
        ],
        out_shape=[
            jax.ShapeDtypeStruct((N, B), jnp.float32),
            jax.ShapeDtypeStruct((N, _K), jnp.int32),
        ],
    )(root_node_feature, neighbor_node_feature, neighbor_edge_feature,
      root_ts[:, None], neighbor_ts, neighbor_nid, gumbel,
      W_node, b_node[None, :], W_edge, b_edge[None, :],
      W_q, b_q[None, :], W_k, b_k[None, :], w[None, :], enc0, enc1)
    return probs, action


# speedup vs baseline: 1.0884x; 1.0884x over previous
import jax
import jax.numpy as jnp
import numpy as np
from jax.experimental import pallas as pl

_N = 10000
_B = 32
_K = 16
_D_NODE = 128
_D_EDGE = 16
_D_FEAT = 32
_D_ENC = 32
_D_HID = 64
_R = 40


def _dot(a, b, prec=None):
    return jax.lax.dot_general(a, b, (((1,), (0,)), ((), ())),
                               precision=prec, preferred_element_type=jnp.float32)


def _lane_bcast(col, n):
    return _dot(col, jnp.ones((1, n), jnp.float32), "highest")


def _fused_kernel(root_ref, neigh_ref, edge_ref, rts_ref, nts_ref, nidc_ref,
                  nidr_ref, gum_ref, Wn_ref, bn_ref, We_ref, be_ref, Wq1_ref,
                  qc_ref, Wka_ref, Wkb_ref, Wkc_ref, Wkd_ref, Wke_ref, bk_ref,
                  w_ref, S_ref, ST_ref, lut_ref, bbf_ref, eyeb_ref,
                  probs_ref, action_ref):
    R, B = _R, _B
    RB = R * B
    S = S_ref[...]

    def _grp(x):
        return _dot(S, x, "highest")

    rf = jnp.maximum(_dot(root_ref[...], Wn_ref[...]) + bn_ref[...], 0.0)
    q = _dot(rf, Wq1_ref[...]) + qc_ref[...]

    nf = jnp.maximum(_dot(neigh_ref[...].reshape(RB, _D_NODE), Wn_ref[...])
                     + bn_ref[...], 0.0)
    ef = jnp.maximum(_dot(edge_ref[...].reshape(RB, _D_EDGE), We_ref[...])
                     + be_ref[...], 0.0)

    nidb = _lane_bcast(nidc_ref[...], B)
    nidj = _grp(nidr_ref[...])
    maskf = jnp.maximum(1.0 - jnp.abs(nidb - nidj), 0.0)

    dtb = _lane_bcast(rts_ref[...] - nts_ref[...], _D_ENC)
    te = jnp.cos(dtb * w_ref[...])
    cntb = _dot(maskf, jnp.ones((B, B), jnp.float32), "highest")
    lane = jax.lax.broadcasted_iota(jnp.int32, (RB, B), 1).astype(jnp.float32)
    onehot_cnt = jnp.maximum(1.0 - jnp.abs(cntb - (lane + 1.0)), 0.0)
    fe = _dot(onehot_cnt, lut_ref[...], "highest")

    k = (_dot(nf, Wka_ref[...]) + _dot(ef, Wkb_ref[...])
         + _dot(te, Wkc_ref[...]) + _dot(fe, Wkd_ref[...])
         + _dot(maskf, Wke_ref[...]) + bk_ref[...])
    k3 = k.reshape(R, B, _D_HID)

    scores = jnp.sum(q[:, None, :] * k3, axis=2) / np.sqrt(float(_D_HID))

    m = jnp.max(scores, axis=1, keepdims=True)
    e = jnp.exp(scores - m)
    probs = e / jnp.sum(e, axis=1, keepdims=True)
    probs_ref[...] = probs

    pert = jnp.log(probs + 1e-20) + gum_ref[...]
    pj = _grp(pert)
    pb = _dot(pj * eyeb_ref[...], jnp.ones((B, B), jnp.float32), "highest")
    bbf = bbf_ref[...]
    beats = jnp.logical_or(pj > pb, jnp.logical_and(pj == pb, lane < bbf))
    rankb = _dot(beats.astype(jnp.float32),
                 jnp.ones((B, B), jnp.float32), "highest")
    sel = jnp.maximum(1.0 - jnp.abs(rankb - lane), 0.0)
    act = _dot(ST_ref[...], sel * bbf, "highest")
    action_ref[...] = act[:, :_K].astype(jnp.int32)


def kernel(root_node_feature, neighbor_node_feature, neighbor_edge_feature,
           root_ts, neighbor_ts, neighbor_nid,
           W_node, b_node, W_edge, b_edge, W_q, b_q, W_k, b_k):
    N, B = neighbor_nid.shape
    RB = _R * B
    w = (1.0 / (10.0 ** jnp.linspace(0.0, 9.0, _D_ENC))).astype(jnp.float32)
    enc0 = jnp.cos(jnp.zeros((1,), jnp.float32)[:, None] * w[None, :])
    enc1 = jnp.cos(jnp.ones((1,), jnp.float32)[:, None] * w[None, :])
    qc = (enc0 @ W_q[_D_FEAT:_D_FEAT + _D_ENC]
          + enc1 @ W_q[_D_FEAT + _D_ENC:] + b_q[None, :])
    Wq1 = W_q[:_D_FEAT]
    Wka = W_k[:_D_FEAT]
    Wkb = W_k[_D_FEAT:2 * _D_FEAT]
    Wkc = W_k[2 * _D_FEAT:2 * _D_FEAT + _D_ENC]
    Wkd = W_k[2 * _D_FEAT + _D_ENC:2 * _D_FEAT + 2 * _D_ENC]
    Wke = W_k[2 * _D_FEAT + 2 * _D_ENC:]
    lut = jnp.cos((jnp.arange(1, B + 1, dtype=jnp.float32) / float(B))[:, None]
                  * w[None, :])

    rows = np.arange(RB)
    S = (rows[:, None] // B == np.arange(_R)[None, :]).astype(np.float32)
    ST = S.T.copy()
    bbf = np.broadcast_to((rows % B).astype(np.float32)[:, None], (RB, B)).copy()
    eyeb = (rows[:, None] % B == np.arange(B)[None, :]).astype(np.float32)

    rts_rep = jnp.repeat(root_ts, B).reshape(N * B, 1)
    nts_flat = neighbor_ts.reshape(N * B, 1)
    nidf = neighbor_nid.astype(jnp.float32)
    nid_col = nidf.reshape(N * B, 1)

    gkey = jax.random.fold_in(jax.random.key(0), 123)
    u = jax.random.uniform(gkey, (N, B), jnp.float32, 1e-10, 1.0)
    gumbel = -jnp.log(-jnp.log(u))

    grid = (N // _R,)
    cst = lambda i: (0, 0)
    probs, action = pl.pallas_call(
        _fused_kernel,
        grid=grid,
        in_specs=[
            pl.BlockSpec((_R, _D_NODE), lambda i: (i, 0)),
            pl.BlockSpec((_R, B, _D_NODE), lambda i: (i, 0, 0)),
            pl.BlockSpec((_R, B, _D_EDGE), lambda i: (i, 0, 0)),
            pl.BlockSpec((RB, 1), lambda i: (i, 0)),
            pl.BlockSpec((RB, 1), lambda i: (i, 0)),
            pl.BlockSpec((RB, 1), lambda i: (i, 0)),
            pl.BlockSpec((_R, B), lambda i: (i, 0)),
            pl.BlockSpec((_R, B), lambda i: (i, 0)),
            pl.BlockSpec((_D_NODE, _D_FEAT), cst),
            pl.BlockSpec((1, _D_FEAT), cst),
            pl.BlockSpec((_D_EDGE, _D_FEAT), cst),
            pl.BlockSpec((1, _D_FEAT), cst),
            pl.BlockSpec((_D_FEAT, _D_HID), cst),
            pl.BlockSpec((1, _D_HID), cst),
            pl.BlockSpec((_D_FEAT, _D_HID), cst),
            pl.BlockSpec((_D_FEAT, _D_HID), cst),
            pl.BlockSpec((_D_ENC, _D_HID), cst),
            pl.BlockSpec((_D_ENC, _D_HID), cst),
            pl.BlockSpec((_D_ENC, _D_HID), cst),
            pl.BlockSpec((1, _D_HID), cst),
            pl.BlockSpec((1, _D_ENC), cst),
            pl.BlockSpec((RB, _R), cst),
            pl.BlockSpec((_R, RB), cst),
            pl.BlockSpec((B, _D_ENC), cst),
            pl.BlockSpec((RB, B), cst),
            pl.BlockSpec((RB, B), cst),
        ],
        out_specs=[
            pl.BlockSpec((_R, B), lambda i: (i, 0)),
            pl.BlockSpec((_R, _K), lambda i: (i, 0)),
        ],
        out_shape=[
            jax.ShapeDtypeStruct((N, B), jnp.float32),
            jax.ShapeDtypeStruct((N, _K), jnp.int32),
        ],
    )(root_node_feature, neighbor_node_feature, neighbor_edge_feature,
      rts_rep, nts_flat, nid_col, nidf, gumbel,
      W_node, b_node[None, :], W_edge, b_edge[None, :],
      Wq1, qc, Wka, Wkb, Wkc, Wkd, Wke, b_k[None, :], w[None, :],
      S, ST, lut, bbf, eyeb)
    return probs, action


# speedup vs baseline: 1.0978x; 1.0086x over previous
import jax
import jax.numpy as jnp
import numpy as np
from jax.experimental import pallas as pl

_N = 10000
_B = 32
_K = 16
_D_NODE = 128
_D_EDGE = 16
_D_FEAT = 32
_D_ENC = 32
_D_HID = 64
_R = 80


def _dot(a, b, prec=None):
    return jax.lax.dot_general(a, b, (((1,), (0,)), ((), ())),
                               precision=prec, preferred_element_type=jnp.float32)


def _lane_bcast(col, n):
    return _dot(col, jnp.ones((1, n), jnp.float32), "highest")


def _fused_kernel(root_ref, neigh_ref, edge_ref, rts_ref, nts_ref, nidc_ref,
                  nidr_ref, gum_ref, Wn_ref, bn_ref, We_ref, be_ref, Wq1_ref,
                  qc_ref, Wka_ref, Wkb_ref, Wkc_ref, Wkd_ref, Wke_ref, bk_ref,
                  w_ref, S_ref, ST_ref, lut_ref, bbf_ref, eyeb_ref,
                  probs_ref, action_ref):
    R, B = _R, _B
    RB = R * B
    S = S_ref[...]

    def _grp(x):
        return _dot(S, x, "highest")

    rf = jnp.maximum(_dot(root_ref[...], Wn_ref[...]) + bn_ref[...], 0.0)
    q = _dot(rf, Wq1_ref[...]) + qc_ref[...]

    nf = jnp.maximum(_dot(neigh_ref[...].reshape(RB, _D_NODE), Wn_ref[...])
                     + bn_ref[...], 0.0)
    ef = jnp.maximum(_dot(edge_ref[...].reshape(RB, _D_EDGE), We_ref[...])
                     + be_ref[...], 0.0)

    nidb = _lane_bcast(nidc_ref[...], B)
    nidj = _grp(nidr_ref[...])
    maskf = jnp.maximum(1.0 - jnp.abs(nidb - nidj), 0.0)

    dtb = _lane_bcast(rts_ref[...] - nts_ref[...], _D_ENC)
    te = jnp.cos(dtb * w_ref[...])
    cntb = _dot(maskf, jnp.ones((B, B), jnp.float32), "highest")
    lane = jax.lax.broadcasted_iota(jnp.int32, (RB, B), 1).astype(jnp.float32)
    onehot_cnt = jnp.maximum(1.0 - jnp.abs(cntb - (lane + 1.0)), 0.0)
    fe = _dot(onehot_cnt, lut_ref[...], "highest")

    k = (_dot(nf, Wka_ref[...]) + _dot(ef, Wkb_ref[...])
         + _dot(te, Wkc_ref[...]) + _dot(fe, Wkd_ref[...])
         + _dot(maskf, Wke_ref[...]) + bk_ref[...])
    k3 = k.reshape(R, B, _D_HID)

    scores = jnp.sum(q[:, None, :] * k3, axis=2) / np.sqrt(float(_D_HID))

    m = jnp.max(scores, axis=1, keepdims=True)
    e = jnp.exp(scores - m)
    probs = e / jnp.sum(e, axis=1, keepdims=True)
    probs_ref[...] = probs

    pert = jnp.log(probs + 1e-20) + gum_ref[...]
    pj = _grp(pert)
    pb = _dot(pj * eyeb_ref[...], jnp.ones((B, B), jnp.float32), "highest")
    bbf = bbf_ref[...]
    beats = jnp.logical_or(pj > pb, jnp.logical_and(pj == pb, lane < bbf))
    rankb = _dot(beats.astype(jnp.float32),
                 jnp.ones((B, B), jnp.float32), "highest")
    sel = jnp.maximum(1.0 - jnp.abs(rankb - lane), 0.0)
    act = _dot(ST_ref[...], sel * bbf, "highest")
    action_ref[...] = act[:, :_K].astype(jnp.int32)


def kernel(root_node_feature, neighbor_node_feature, neighbor_edge_feature,
           root_ts, neighbor_ts, neighbor_nid,
           W_node, b_node, W_edge, b_edge, W_q, b_q, W_k, b_k):
    N, B = neighbor_nid.shape
    RB = _R * B
    w = (1.0 / (10.0 ** jnp.linspace(0.0, 9.0, _D_ENC))).astype(jnp.float32)
    enc0 = jnp.cos(jnp.zeros((1,), jnp.float32)[:, None] * w[None, :])
    enc1 = jnp.cos(jnp.ones((1,), jnp.float32)[:, None] * w[None, :])
    qc = (enc0 @ W_q[_D_FEAT:_D_FEAT + _D_ENC]
          + enc1 @ W_q[_D_FEAT + _D_ENC:] + b_q[None, :])
    Wq1 = W_q[:_D_FEAT]
    Wka = W_k[:_D_FEAT]
    Wkb = W_k[_D_FEAT:2 * _D_FEAT]
    Wkc = W_k[2 * _D_FEAT:2 * _D_FEAT + _D_ENC]
    Wkd = W_k[2 * _D_FEAT + _D_ENC:2 * _D_FEAT + 2 * _D_ENC]
    Wke = W_k[2 * _D_FEAT + 2 * _D_ENC:]
    lut = jnp.cos((jnp.arange(1, B + 1, dtype=jnp.float32) / float(B))[:, None]
                  * w[None, :])

    rows = np.arange(RB)
    S = (rows[:, None] // B == np.arange(_R)[None, :]).astype(np.float32)
    ST = S.T.copy()
    bbf = np.broadcast_to((rows % B).astype(np.float32)[:, None], (RB, B)).copy()
    eyeb = (rows[:, None] % B == np.arange(B)[None, :]).astype(np.float32)

    rts_rep = jnp.repeat(root_ts, B).reshape(N * B, 1)
    nts_flat = neighbor_ts.reshape(N * B, 1)
    nidf = neighbor_nid.astype(jnp.float32)
    nid_col = nidf.reshape(N * B, 1)

    gkey = jax.random.fold_in(jax.random.key(0), 123)
    u = jax.random.uniform(gkey, (N, B), jnp.float32, 1e-10, 1.0)
    gumbel = -jnp.log(-jnp.log(u))

    grid = (N // _R,)
    cst = lambda i: (0, 0)
    probs, action = pl.pallas_call(
        _fused_kernel,
        grid=grid,
        in_specs=[
            pl.BlockSpec((_R, _D_NODE), lambda i: (i, 0)),
            pl.BlockSpec((_R, B, _D_NODE), lambda i: (i, 0, 0)),
            pl.BlockSpec((_R, B, _D_EDGE), lambda i: (i, 0, 0)),
            pl.BlockSpec((RB, 1), lambda i: (i, 0)),
            pl.BlockSpec((RB, 1), lambda i: (i, 0)),
            pl.BlockSpec((RB, 1), lambda i: (i, 0)),
            pl.BlockSpec((_R, B), lambda i: (i, 0)),
            pl.BlockSpec((_R, B), lambda i: (i, 0)),
            pl.BlockSpec((_D_NODE, _D_FEAT), cst),
            pl.BlockSpec((1, _D_FEAT), cst),
            pl.BlockSpec((_D_EDGE, _D_FEAT), cst),
            pl.BlockSpec((1, _D_FEAT), cst),
            pl.BlockSpec((_D_FEAT, _D_HID), cst),
            pl.BlockSpec((1, _D_HID), cst),
            pl.BlockSpec((_D_FEAT, _D_HID), cst),
            pl.BlockSpec((_D_FEAT, _D_HID), cst),
            pl.BlockSpec((_D_ENC, _D_HID), cst),
            pl.BlockSpec((_D_ENC, _D_HID), cst),
            pl.BlockSpec((_D_ENC, _D_HID), cst),
            pl.BlockSpec((1, _D_HID), cst),
            pl.BlockSpec((1, _D_ENC), cst),
            pl.BlockSpec((RB, _R), cst),
            pl.BlockSpec((_R, RB), cst),
            pl.BlockSpec((B, _D_ENC), cst),
            pl.BlockSpec((RB, B), cst),
            pl.BlockSpec((RB, B), cst),
        ],
        out_specs=[
            pl.BlockSpec((_R, B), lambda i: (i, 0)),
            pl.BlockSpec((_R, _K), lambda i: (i, 0)),
        ],
        out_shape=[
            jax.ShapeDtypeStruct((N, B), jnp.float32),
            jax.ShapeDtypeStruct((N, _K), jnp.int32),
        ],
    )(root_node_feature, neighbor_node_feature, neighbor_edge_feature,
      rts_rep, nts_flat, nid_col, nidf, gumbel,
      W_node, b_node[None, :], W_edge, b_edge[None, :],
      Wq1, qc, Wka, Wkb, Wkc, Wkd, Wke, b_k[None, :], w[None, :],
      S, ST, lut, bbf, eyeb)
    return probs, action


# speedup vs baseline: 2.0126x; 1.8334x over previous
import jax
import jax.numpy as jnp
import numpy as np
from jax.experimental import pallas as pl

_N = 10000
_B = 32
_K = 16
_D_NODE = 128
_D_EDGE = 16
_D_FEAT = 32
_D_ENC = 32
_D_HID = 64
_R = 40


def _dot(a, b, prec=None, dims=(((1,), (0,)), ((), ()))):
    return jax.lax.dot_general(a, b, dims,
                               precision=prec, preferred_element_type=jnp.float32)


def _dot_rt(a, b, prec=None):
    return jax.lax.dot_general(a, b, (((1,), (1,)), ((), ())),
                               precision=prec, preferred_element_type=jnp.float32)


def _dot_lt(a, b, prec=None):
    return jax.lax.dot_general(a, b, (((0,), (0,)), ((), ())),
                               precision=prec, preferred_element_type=jnp.float32)


def _lane_bcast(col, n):
    return _dot(col, jnp.ones((1, n), jnp.float32), "highest")


def _fused_kernel(root_ref, neigh_ref, edge_ref, rts_ref, nts_ref, nid1_ref,
                  nidT_ref, gum_ref, WnT_ref, bnR_ref, bnT_ref, WeT_ref,
                  beT_ref, Wq1_ref, qc_ref, Wka_ref, Wkb_ref, Wkc_ref,
                  Wkd_ref, Wke_ref, bk_ref, wt_ref, rc_ref, lutT_ref,
                  S_ref, ST_ref, bbf_ref, eyeb_ref, probs_ref, action_ref):
    R, B = _R, _B
    RB = R * B
    ST = ST_ref[...]

    rfT = jnp.maximum(_dot_rt(WnT_ref[...], root_ref[...]) + bnR_ref[...], 0.0)
    q = _dot_lt(rfT, Wq1_ref[...]) + qc_ref[...]

    nfT = jnp.maximum(_dot_rt(WnT_ref[...], neigh_ref[...].reshape(RB, _D_NODE))
                      + bnT_ref[...], 0.0)
    efT = jnp.maximum(_dot_rt(WeT_ref[...], edge_ref[...].reshape(RB, _D_EDGE))
                      + beT_ref[...], 0.0)

    nidbT = jnp.broadcast_to(nid1_ref[...], (B, RB))
    nidjT = _dot(nidT_ref[0], ST, "highest")
    maskT = jnp.maximum(1.0 - jnp.abs(nidbT - nidjT), 0.0)

    dtT = jnp.broadcast_to(rts_ref[...] - nts_ref[...], (_D_ENC, RB))
    teT = jnp.cos(dtT * wt_ref[...])
    cntT = _dot(jnp.ones((B, B), jnp.float32), maskT, "highest")
    ohT = jnp.maximum(1.0 - jnp.abs(cntT - rc_ref[...]), 0.0)
    feT = _dot(lutT_ref[...], ohT, "highest")

    k = (_dot_lt(nfT, Wka_ref[...]) + _dot_lt(efT, Wkb_ref[...])
         + _dot_lt(teT, Wkc_ref[...]) + _dot_lt(feT, Wkd_ref[...])
         + _dot_lt(maskT, Wke_ref[...]) + bk_ref[...])
    k3 = k.reshape(R, B, _D_HID)

    scores = jnp.sum(q[:, None, :] * k3, axis=2) / np.sqrt(float(_D_HID))

    m = jnp.max(scores, axis=1, keepdims=True)
    e = jnp.exp(scores - m)
    probs = e / jnp.sum(e, axis=1, keepdims=True)
    probs_ref[...] = probs

    pert = jnp.log(probs + 1e-20) + gum_ref[...]
    pj = _dot(S_ref[...], pert, "highest")
    pb = _dot(pj * eyeb_ref[...], jnp.ones((B, B), jnp.float32), "highest")
    bbf = bbf_ref[...]
    lane = jax.lax.broadcasted_iota(jnp.int32, (RB, B), 1).astype(jnp.float32)
    beats = jnp.logical_or(pj > pb, jnp.logical_and(pj == pb, lane < bbf))
    rankb = _dot(beats.astype(jnp.float32),
                 jnp.ones((B, B), jnp.float32), "highest")
    sel = jnp.maximum(1.0 - jnp.abs(rankb - lane), 0.0)
    act = _dot(ST, sel * bbf, "highest")
    action_ref[...] = act[:, :_K].astype(jnp.int32)


def kernel(root_node_feature, neighbor_node_feature, neighbor_edge_feature,
           root_ts, neighbor_ts, neighbor_nid,
           W_node, b_node, W_edge, b_edge, W_q, b_q, W_k, b_k):
    N, B = neighbor_nid.shape
    RB = _R * B
    w = (1.0 / (10.0 ** jnp.linspace(0.0, 9.0, _D_ENC))).astype(jnp.float32)
    enc0 = jnp.cos(jnp.zeros((1,), jnp.float32)[:, None] * w[None, :])
    enc1 = jnp.cos(jnp.ones((1,), jnp.float32)[:, None] * w[None, :])
    qc = (enc0 @ W_q[_D_FEAT:_D_FEAT + _D_ENC]
          + enc1 @ W_q[_D_FEAT + _D_ENC:] + b_q[None, :])
    lut = jnp.cos((jnp.arange(1, B + 1, dtype=jnp.float32) / float(B))[:, None]
                  * w[None, :])

    WnT = W_node.T
    WeT = W_edge.T
    Wq1 = W_q[:_D_FEAT]
    Wka = W_k[:_D_FEAT]
    Wkb = W_k[_D_FEAT:2 * _D_FEAT]
    Wkc = W_k[2 * _D_FEAT:2 * _D_FEAT + _D_ENC]
    Wkd = W_k[2 * _D_FEAT + _D_ENC:2 * _D_FEAT + 2 * _D_ENC]
    Wke = W_k[2 * _D_FEAT + 2 * _D_ENC:]
    lutT = lut.T
    bnR = jnp.broadcast_to(b_node[:, None], (_D_FEAT, _R))
    bnT = jnp.broadcast_to(b_node[:, None], (_D_FEAT, RB))
    beT = jnp.broadcast_to(b_edge[:, None], (_D_FEAT, RB))
    wt = jnp.broadcast_to(w[:, None], (_D_ENC, RB))
    rc = jnp.broadcast_to(jnp.arange(1, B + 1, dtype=jnp.float32)[:, None],
                          (B, RB))

    rows = np.arange(RB)
    S = (rows[:, None] // B == np.arange(_R)[None, :]).astype(np.float32)
    ST = S.T.copy()
    bbf = np.broadcast_to((rows % B).astype(np.float32)[:, None], (RB, B)).copy()
    eyeb = (rows[:, None] % B == np.arange(B)[None, :]).astype(np.float32)

    rts_rep = jnp.repeat(root_ts, B).reshape(1, N * B)
    nts_flat = neighbor_ts.reshape(1, N * B)
    nidf = neighbor_nid.astype(jnp.float32)
    nid_row = nidf.reshape(1, N * B)
    nidT = nidf.T.reshape(B, N // _R, _R).transpose(1, 0, 2)

    gkey = jax.random.fold_in(jax.random.key(0), 123)
    u = jax.random.uniform(gkey, (N, B), jnp.float32, 1e-10, 1.0)
    gumbel = -jnp.log(-jnp.log(u))

    grid = (N // _R,)
    cst = lambda i: (0, 0)
    probs, action = pl.pallas_call(
        _fused_kernel,
        grid=grid,
        in_specs=[
            pl.BlockSpec((_R, _D_NODE), lambda i: (i, 0)),
            pl.BlockSpec((_R, B, _D_NODE), lambda i: (i, 0, 0)),
            pl.BlockSpec((_R, B, _D_EDGE), lambda i: (i, 0, 0)),
            pl.BlockSpec((1, RB), lambda i: (0, i)),
            pl.BlockSpec((1, RB), lambda i: (0, i)),
            pl.BlockSpec((1, RB), lambda i: (0, i)),
            pl.BlockSpec((1, B, _R), lambda i: (i, 0, 0)),
            pl.BlockSpec((_R, B), lambda i: (i, 0)),
            pl.BlockSpec((_D_FEAT, _D_NODE), cst),
            pl.BlockSpec((_D_FEAT, _R), cst),
            pl.BlockSpec((_D_FEAT, RB), cst),
            pl.BlockSpec((_D_FEAT, _D_EDGE), cst),
            pl.BlockSpec((_D_FEAT, RB), cst),
            pl.BlockSpec((_D_FEAT, _D_HID), cst),
            pl.BlockSpec((1, _D_HID), cst),
            pl.BlockSpec((_D_FEAT, _D_HID), cst),
            pl.BlockSpec((_D_FEAT, _D_HID), cst),
            pl.BlockSpec((_D_ENC, _D_HID), cst),
            pl.BlockSpec((_D_ENC, _D_HID), cst),
            pl.BlockSpec((_D_FEAT, _D_HID), cst),
            pl.BlockSpec((1, _D_HID), cst),
            pl.BlockSpec((_D_ENC, RB), cst),
            pl.BlockSpec((B, RB), cst),
            pl.BlockSpec((_D_ENC, B), cst),
            pl.BlockSpec((RB, _R), cst),
            pl.BlockSpec((_R, RB), cst),
            pl.BlockSpec((RB, B), cst),
            pl.BlockSpec((RB, B), cst),
        ],
        out_specs=[
            pl.BlockSpec((_R, B), lambda i: (i, 0)),
            pl.BlockSpec((_R, _K), lambda i: (i, 0)),
        ],
        out_shape=[
            jax.ShapeDtypeStruct((N, B), jnp.float32),
            jax.ShapeDtypeStruct((N, _K), jnp.int32),
        ],
    )(root_node_feature, neighbor_node_feature, neighbor_edge_feature,
      rts_rep, nts_flat, nid_row, nidT, gumbel,
      WnT, bnR, bnT, WeT, beT, Wq1, qc,
      Wka, Wkb, Wkc, Wkd, Wke, b_k[None, :], wt, rc, lutT,
      S, ST, bbf, eyeb)
    return probs, action


# speedup vs baseline: 2.0587x; 1.0229x over previous
import jax
import jax.numpy as jnp
import numpy as np
from jax.experimental import pallas as pl

_N = 10000
_B = 32
_K = 16
_D_NODE = 128
_D_EDGE = 16
_D_FEAT = 32
_D_ENC = 32
_D_HID = 64
_R = 80


def _dot(a, b, prec=None, dims=(((1,), (0,)), ((), ()))):
    return jax.lax.dot_general(a, b, dims,
                               precision=prec, preferred_element_type=jnp.float32)


def _dot_rt(a, b, prec=None):
    return jax.lax.dot_general(a, b, (((1,), (1,)), ((), ())),
                               precision=prec, preferred_element_type=jnp.float32)


def _dot_lt(a, b, prec=None):
    return jax.lax.dot_general(a, b, (((0,), (0,)), ((), ())),
                               precision=prec, preferred_element_type=jnp.float32)


def _lane_bcast(col, n):
    return _dot(col, jnp.ones((1, n), jnp.float32), "highest")


def _fused_kernel(root_ref, neigh_ref, edge_ref, rts_ref, nts_ref, nid1_ref,
                  nidT_ref, gum_ref, WnT_ref, bnR_ref, bnT_ref, WeT_ref,
                  beT_ref, Wq1_ref, qc_ref, Wka_ref, Wkb_ref, Wkc_ref,
                  Wkd_ref, Wke_ref, bk_ref, wt_ref, rc_ref, lutT_ref,
                  S_ref, ST_ref, bbf_ref, eyeb_ref, probs_ref, action_ref):
    R, B = _R, _B
    RB = R * B
    ST = ST_ref[...]

    rfT = jnp.maximum(_dot_rt(WnT_ref[...], root_ref[...]) + bnR_ref[...], 0.0)
    q = _dot_lt(rfT, Wq1_ref[...]) + qc_ref[...]

    nfT = jnp.maximum(_dot_rt(WnT_ref[...], neigh_ref[...].reshape(RB, _D_NODE))
                      + bnT_ref[...], 0.0)
    efT = jnp.maximum(_dot_rt(WeT_ref[...], edge_ref[...].reshape(RB, _D_EDGE))
                      + beT_ref[...], 0.0)

    nidbT = jnp.broadcast_to(nid1_ref[...], (B, RB))
    nidjT = _dot(nidT_ref[0], ST, "highest")
    maskT = jnp.maximum(1.0 - jnp.abs(nidbT - nidjT), 0.0)

    dtT = jnp.broadcast_to(rts_ref[...] - nts_ref[...], (_D_ENC, RB))
    teT = jnp.cos(dtT * wt_ref[...])
    cntT = _dot(jnp.ones((B, B), jnp.float32), maskT, "highest")
    ohT = jnp.maximum(1.0 - jnp.abs(cntT - rc_ref[...]), 0.0)
    feT = _dot(lutT_ref[...], ohT, "highest")

    k = (_dot_lt(nfT, Wka_ref[...]) + _dot_lt(efT, Wkb_ref[...])
         + _dot_lt(teT, Wkc_ref[...]) + _dot_lt(feT, Wkd_ref[...])
         + _dot_lt(maskT, Wke_ref[...]) + bk_ref[...])
    k3 = k.reshape(R, B, _D_HID)

    scores = jnp.sum(q[:, None, :] * k3, axis=2) / np.sqrt(float(_D_HID))

    m = jnp.max(scores, axis=1, keepdims=True)
    e = jnp.exp(scores - m)
    probs = e / jnp.sum(e, axis=1, keepdims=True)
    probs_ref[...] = probs

    pert = jnp.log(probs + 1e-20) + gum_ref[...]
    pj = _dot(S_ref[...], pert, "highest")
    pb = _dot(pj * eyeb_ref[...], jnp.ones((B, B), jnp.float32), "highest")
    bbf = bbf_ref[...]
    lane = jax.lax.broadcasted_iota(jnp.int32, (RB, B), 1).astype(jnp.float32)
    beats = jnp.logical_or(pj > pb, jnp.logical_and(pj == pb, lane < bbf))
    rankb = _dot(beats.astype(jnp.float32),
                 jnp.ones((B, B), jnp.float32), "highest")
    sel = jnp.maximum(1.0 - jnp.abs(rankb - lane), 0.0)
    act = _dot(ST, sel * bbf, "highest")
    action_ref[...] = act[:, :_K].astype(jnp.int32)


def kernel(root_node_feature, neighbor_node_feature, neighbor_edge_feature,
           root_ts, neighbor_ts, neighbor_nid,
           W_node, b_node, W_edge, b_edge, W_q, b_q, W_k, b_k):
    N, B = neighbor_nid.shape
    RB = _R * B
    w = (1.0 / (10.0 ** jnp.linspace(0.0, 9.0, _D_ENC))).astype(jnp.float32)
    enc0 = jnp.cos(jnp.zeros((1,), jnp.float32)[:, None] * w[None, :])
    enc1 = jnp.cos(jnp.ones((1,), jnp.float32)[:, None] * w[None, :])
    qc = (enc0 @ W_q[_D_FEAT:_D_FEAT + _D_ENC]
          + enc1 @ W_q[_D_FEAT + _D_ENC:] + b_q[None, :])
    lut = jnp.cos((jnp.arange(1, B + 1, dtype=jnp.float32) / float(B))[:, None]
                  * w[None, :])

    WnT = W_node.T
    WeT = W_edge.T
    Wq1 = W_q[:_D_FEAT]
    Wka = W_k[:_D_FEAT]
    Wkb = W_k[_D_FEAT:2 * _D_FEAT]
    Wkc = W_k[2 * _D_FEAT:2 * _D_FEAT + _D_ENC]
    Wkd = W_k[2 * _D_FEAT + _D_ENC:2 * _D_FEAT + 2 * _D_ENC]
    Wke = W_k[2 * _D_FEAT + 2 * _D_ENC:]
    lutT = lut.T
    bnR = jnp.broadcast_to(b_node[:, None], (_D_FEAT, _R))
    bnT = jnp.broadcast_to(b_node[:, None], (_D_FEAT, RB))
    beT = jnp.broadcast_to(b_edge[:, None], (_D_FEAT, RB))
    wt = jnp.broadcast_to(w[:, None], (_D_ENC, RB))
    rc = jnp.broadcast_to(jnp.arange(1, B + 1, dtype=jnp.float32)[:, None],
                          (B, RB))

    rows = np.arange(RB)
    S = (rows[:, None] // B == np.arange(_R)[None, :]).astype(np.float32)
    ST = S.T.copy()
    bbf = np.broadcast_to((rows % B).astype(np.float32)[:, None], (RB, B)).copy()
    eyeb = (rows[:, None] % B == np.arange(B)[None, :]).astype(np.float32)

    rts_rep = jnp.repeat(root_ts, B).reshape(1, N * B)
    nts_flat = neighbor_ts.reshape(1, N * B)
    nidf = neighbor_nid.astype(jnp.float32)
    nid_row = nidf.reshape(1, N * B)
    nidT = nidf.T.reshape(B, N // _R, _R).transpose(1, 0, 2)

    gkey = jax.random.fold_in(jax.random.key(0), 123)
    u = jax.random.uniform(gkey, (N, B), jnp.float32, 1e-10, 1.0)
    gumbel = -jnp.log(-jnp.log(u))

    grid = (N // _R,)
    cst = lambda i: (0, 0)
    probs, action = pl.pallas_call(
        _fused_kernel,
        grid=grid,
        in_specs=[
            pl.BlockSpec((_R, _D_NODE), lambda i: (i, 0)),
            pl.BlockSpec((_R, B, _D_NODE), lambda i: (i, 0, 0)),
            pl.BlockSpec((_R, B, _D_EDGE), lambda i: (i, 0, 0)),
            pl.BlockSpec((1, RB), lambda i: (0, i)),
            pl.BlockSpec((1, RB), lambda i: (0, i)),
            pl.BlockSpec((1, RB), lambda i: (0, i)),
            pl.BlockSpec((1, B, _R), lambda i: (i, 0, 0)),
            pl.BlockSpec((_R, B), lambda i: (i, 0)),
            pl.BlockSpec((_D_FEAT, _D_NODE), cst),
            pl.BlockSpec((_D_FEAT, _R), cst),
            pl.BlockSpec((_D_FEAT, RB), cst),
            pl.BlockSpec((_D_FEAT, _D_EDGE), cst),
            pl.BlockSpec((_D_FEAT, RB), cst),
            pl.BlockSpec((_D_FEAT, _D_HID), cst),
            pl.BlockSpec((1, _D_HID), cst),
            pl.BlockSpec((_D_FEAT, _D_HID), cst),
            pl.BlockSpec((_D_FEAT, _D_HID), cst),
            pl.BlockSpec((_D_ENC, _D_HID), cst),
            pl.BlockSpec((_D_ENC, _D_HID), cst),
            pl.BlockSpec((_D_FEAT, _D_HID), cst),
            pl.BlockSpec((1, _D_HID), cst),
            pl.BlockSpec((_D_ENC, RB), cst),
            pl.BlockSpec((B, RB), cst),
            pl.BlockSpec((_D_ENC, B), cst),
            pl.BlockSpec((RB, _R), cst),
            pl.BlockSpec((_R, RB), cst),
            pl.BlockSpec((RB, B), cst),
            pl.BlockSpec((RB, B), cst),
        ],
        out_specs=[
            pl.BlockSpec((_R, B), lambda i: (i, 0)),
            pl.BlockSpec((_R, _K), lambda i: (i, 0)),
        ],
        out_shape=[
            jax.ShapeDtypeStruct((N, B), jnp.float32),
            jax.ShapeDtypeStruct((N, _K), jnp.int32),
        ],
    )(root_node_feature, neighbor_node_feature, neighbor_edge_feature,
      rts_rep, nts_flat, nid_row, nidT, gumbel,
      WnT, bnR, bnT, WeT, beT, Wq1, qc,
      Wka, Wkb, Wkc, Wkd, Wke, b_k[None, :], wt, rc, lutT,
      S, ST, bbf, eyeb)
    return probs, action
